# baseline (speedup 1.0000x reference)
import functools

import jax
import jax.numpy as jnp
from jax import lax
from jax.experimental import pallas as pl
from jax.experimental.pallas import tpu as pltpu
from jax.experimental.pallas import tpu_sc as plsc

M_ROWS = 1_000_000
DIM = 16
BATCH = 16384
LR = 0.01

NUM_CORES = 2
NUM_SUBCORES = 16
LANES = 16

NUM_PASSES = 8
ROWS_PER_CORE = M_ROWS // NUM_CORES
ROWS_PER_PASS = ROWS_PER_CORE // NUM_PASSES
DUMMY_SLOT = ROWS_PER_PASS
PAD_SPREAD = 128
DUMMY_SPREAD = 8
B_PER_TILE = BATCH // NUM_SUBCORES
CHUNK = 128
NCHUNK = B_PER_TILE // CHUNK


def _copy_body(x_ref, o_ref):
    o_ref[...] = x_ref[...]


_COPY_ROWS = 5000


def _tc_copy(x):
    x2 = x.reshape(M_ROWS * DIM // 128, 128)
    out = pl.pallas_call(
        _copy_body,
        grid=(x2.shape[0] // _COPY_ROWS,),
        in_specs=[pl.BlockSpec((_COPY_ROWS, 128), lambda i: (i, 0))],
        out_specs=pl.BlockSpec((_COPY_ROWS, 128), lambda i: (i, 0)),
        out_shape=jax.ShapeDtypeStruct(x2.shape, x.dtype),
    )(x2)
    return out.reshape(M_ROWS, DIM)


COPY_PER_TILE = ROWS_PER_PASS // NUM_SUBCORES
COPY_REMAINDER = ROWS_PER_PASS - COPY_PER_TILE * NUM_SUBCORES


def _sc_body(emb_hbm, idx_hbm, val_hbm, out_hbm,
             spmem, idx_v, vbuf, ebuf, zidx, aidx, oidx, sem, csem,
             _axis_index=lax.axis_index):
    c = _axis_index("c")
    s = _axis_index("s")
    base = s * B_PER_TILE

    pltpu.sync_copy(idx_hbm.at[pl.ds(base, B_PER_TILE)], idx_v)
    pltpu.sync_copy(val_hbm.at[pl.ds(base, B_PER_TILE)], vbuf)

    def start_range_copy(p):
        lo_p = (c * NUM_PASSES + p) * ROWS_PER_PASS
        cbase = lo_p + s * COPY_PER_TILE
        cps = [pltpu.async_copy(emb_hbm.at[pl.ds(cbase, COPY_PER_TILE)],
                                out_hbm.at[pl.ds(cbase, COPY_PER_TILE)], csem)]
        tail = lo_p + COPY_PER_TILE * NUM_SUBCORES
        cps.append(pltpu.async_copy(
            emb_hbm.at[pl.ds(tail, COPY_REMAINDER)],
            out_hbm.at[pl.ds(tail, COPY_REMAINDER)], csem))
        return cps

    copies_p = start_range_copy(0)

    @pl.loop(0, B_PER_TILE)
    def _(g):
        vbuf[g, :] = vbuf[g, :] * (-LR)

    for p in range(NUM_PASSES):
        lo = (c * NUM_PASSES + p) * ROWS_PER_PASS

        for j in range(NCHUNK):
            @pl.loop(0, CHUNK // LANES)
            def _(k, j=j):
                i16 = idx_v[pl.ds(j * CHUNK + k * LANES, LANES)]
                inr = (i16 >= lo) & (i16 < lo + ROWS_PER_PASS)
                l16 = i16 - lo
                pos = lax.iota(jnp.int32, LANES) + (j * CHUNK + k * LANES)
                spread = pos & (PAD_SPREAD - 1)
                zidx[j][pl.ds(k * LANES, LANES)] = jnp.where(inr, l16, spread)
                aidx[j][pl.ds(k * LANES, LANES)] = jnp.where(
                    inr, l16, DUMMY_SLOT + (pos & (DUMMY_SPREAD - 1)))
                oidx[j][pl.ds(k * LANES, LANES)] = jnp.where(
                    inr, i16, lo + spread)

        gcopies = [
            pltpu.async_copy(emb_hbm.at[oidx[j]],
                             ebuf.at[pl.ds(j * CHUNK, CHUNK)], sem)
            for j in range(NCHUNK)
        ]
        for cp in gcopies:
            cp.wait()
        scopies = [
            pltpu.async_copy(ebuf.at[pl.ds(j * CHUNK, CHUNK)],
                             spmem.at[zidx[j]], sem)
            for j in range(NCHUNK)
        ]
        for cp in scopies:
            cp.wait()
        for cp in copies_p:
            cp.wait()
        plsc.subcore_barrier()
        if p + 1 < NUM_PASSES:
            copies_p = start_range_copy(p + 1)

        acopies = [
            pltpu.async_copy(vbuf.at[pl.ds(j * CHUNK, CHUNK)],
                             spmem.at[aidx[j]], sem, add=True)
            for j in range(NCHUNK)
        ]
        for cp in acopies:
            cp.wait()
        plsc.subcore_barrier()

        rcopies = [
            pltpu.async_copy(spmem.at[zidx[j]],
                             ebuf.at[pl.ds(j * CHUNK, CHUNK)], sem)
            for j in range(NCHUNK)
        ]
        for cp in rcopies:
            cp.wait()
        wcopies = [
            pltpu.async_copy(ebuf.at[pl.ds(j * CHUNK, CHUNK)],
                             out_hbm.at[oidx[j]], sem)
            for j in range(NCHUNK)
        ]
        for cp in wcopies:
            cp.wait()
        plsc.subcore_barrier()


@functools.cache
def _get_sc_update(interpret=False, _axis_index=lax.axis_index):
    return pl.kernel(
        functools.partial(_sc_body, _axis_index=_axis_index),
        out_type=jax.ShapeDtypeStruct((M_ROWS, DIM), jnp.float32),
        mesh=plsc.VectorSubcoreMesh(core_axis_name="c", subcore_axis_name="s",
                                    num_cores=NUM_CORES,
                                    num_subcores=NUM_SUBCORES),
        scratch_types=[
            pltpu.VMEM_SHARED((ROWS_PER_PASS + 8, DIM), jnp.float32),
            pltpu.VMEM((B_PER_TILE,), jnp.int32),
            pltpu.VMEM((B_PER_TILE, DIM), jnp.float32),
            pltpu.VMEM((B_PER_TILE, DIM), jnp.float32),
            [pltpu.VMEM((CHUNK,), jnp.int32) for _ in range(NCHUNK)],
            [pltpu.VMEM((CHUNK,), jnp.int32) for _ in range(NCHUNK)],
            [pltpu.VMEM((CHUNK,), jnp.int32) for _ in range(NCHUNK)],
            pltpu.SemaphoreType.DMA,
            pltpu.SemaphoreType.DMA,
        ],
        compiler_params=pltpu.CompilerParams(use_tc_tiling_on_sc=False),
        interpret=interpret,
    )


def kernel(items_emb, idx, val):
    return _get_sc_update()(items_emb, idx, val)


# speedup vs baseline: 2.7263x; 2.7263x over previous
import functools

import jax
import jax.numpy as jnp
from jax import lax
from jax.experimental import pallas as pl
from jax.experimental.pallas import tpu as pltpu
from jax.experimental.pallas import tpu_sc as plsc

M_ROWS = 1_000_000
DIM = 16
BATCH = 16384
LR = 0.01

NUM_CORES = 2
NUM_SUBCORES = 16
LANES = 16

NUM_PASSES = 8
ROWS_PER_CORE = M_ROWS // NUM_CORES
ROWS_PER_PASS = ROWS_PER_CORE // NUM_PASSES
DUMMY_SLOT = ROWS_PER_PASS
PAD_SPREAD = 128
DUMMY_SPREAD = 8
B_PER_TILE = BATCH // NUM_SUBCORES
CHUNK = 128
NCHUNK = B_PER_TILE // CHUNK


def _copy_body(x_ref, o_ref):
    o_ref[...] = x_ref[...]


_COPY_ROWS = 5000


def _tc_copy(x):
    x2 = x.reshape(M_ROWS * DIM // 128, 128)
    out = pl.pallas_call(
        _copy_body,
        grid=(x2.shape[0] // _COPY_ROWS,),
        in_specs=[pl.BlockSpec((_COPY_ROWS, 128), lambda i: (i, 0))],
        out_specs=pl.BlockSpec((_COPY_ROWS, 128), lambda i: (i, 0)),
        out_shape=jax.ShapeDtypeStruct(x2.shape, x.dtype),
    )(x2)
    return out.reshape(M_ROWS, DIM)


def _sc_body(idx_hbm, val_hbm, out_hbm,
             spmem, idx_v, vbuf, ebuf, zidx, aidx, oidx, sem,
             _axis_index=lax.axis_index):
    c = _axis_index("c")
    s = _axis_index("s")
    base = s * B_PER_TILE

    pltpu.sync_copy(idx_hbm.at[pl.ds(base, B_PER_TILE)], idx_v)
    pltpu.sync_copy(val_hbm.at[pl.ds(base, B_PER_TILE)], vbuf)

    @pl.loop(0, B_PER_TILE)
    def _(g):
        vbuf[g, :] = vbuf[g, :] * (-LR)

    for p in range(NUM_PASSES):
        lo = (c * NUM_PASSES + p) * ROWS_PER_PASS

        for j in range(NCHUNK):
            @pl.loop(0, CHUNK // LANES)
            def _(k, j=j):
                i16 = idx_v[pl.ds(j * CHUNK + k * LANES, LANES)]
                inr = (i16 >= lo) & (i16 < lo + ROWS_PER_PASS)
                l16 = i16 - lo
                pos = lax.iota(jnp.int32, LANES) + (j * CHUNK + k * LANES)
                spread = pos & (PAD_SPREAD - 1)
                zidx[j][pl.ds(k * LANES, LANES)] = jnp.where(inr, l16, spread)
                aidx[j][pl.ds(k * LANES, LANES)] = jnp.where(
                    inr, l16, DUMMY_SLOT + (pos & (DUMMY_SPREAD - 1)))
                oidx[j][pl.ds(k * LANES, LANES)] = jnp.where(
                    inr, i16, lo + spread)

        gcopies = [
            pltpu.async_copy(out_hbm.at[oidx[j]],
                             ebuf.at[pl.ds(j * CHUNK, CHUNK)], sem)
            for j in range(NCHUNK)
        ]
        for cp in gcopies:
            cp.wait()
        scopies = [
            pltpu.async_copy(ebuf.at[pl.ds(j * CHUNK, CHUNK)],
                             spmem.at[zidx[j]], sem)
            for j in range(NCHUNK)
        ]
        for cp in scopies:
            cp.wait()
        plsc.subcore_barrier()

        acopies = [
            pltpu.async_copy(vbuf.at[pl.ds(j * CHUNK, CHUNK)],
                             spmem.at[aidx[j]], sem, add=True)
            for j in range(NCHUNK)
        ]
        for cp in acopies:
            cp.wait()
        plsc.subcore_barrier()

        rcopies = [
            pltpu.async_copy(spmem.at[zidx[j]],
                             ebuf.at[pl.ds(j * CHUNK, CHUNK)], sem)
            for j in range(NCHUNK)
        ]
        for cp in rcopies:
            cp.wait()
        wcopies = [
            pltpu.async_copy(ebuf.at[pl.ds(j * CHUNK, CHUNK)],
                             out_hbm.at[oidx[j]], sem)
            for j in range(NCHUNK)
        ]
        for cp in wcopies:
            cp.wait()
        plsc.subcore_barrier()


@functools.cache
def _get_sc_update(interpret=False, _axis_index=lax.axis_index):
    return pl.kernel(
        functools.partial(_sc_body, _axis_index=_axis_index),
        out_type=(),
        mesh=plsc.VectorSubcoreMesh(core_axis_name="c", subcore_axis_name="s",
                                    num_cores=NUM_CORES,
                                    num_subcores=NUM_SUBCORES),
        scratch_types=[
            pltpu.VMEM_SHARED((ROWS_PER_PASS + 8, DIM), jnp.float32),
            pltpu.VMEM((B_PER_TILE,), jnp.int32),
            pltpu.VMEM((B_PER_TILE, DIM), jnp.float32),
            pltpu.VMEM((B_PER_TILE, DIM), jnp.float32),
            [pltpu.VMEM((CHUNK,), jnp.int32) for _ in range(NCHUNK)],
            [pltpu.VMEM((CHUNK,), jnp.int32) for _ in range(NCHUNK)],
            [pltpu.VMEM((CHUNK,), jnp.int32) for _ in range(NCHUNK)],
            pltpu.SemaphoreType.DMA,
        ],
        compiler_params=pltpu.CompilerParams(use_tc_tiling_on_sc=False),
        interpret=interpret,
    )


def kernel(items_emb, idx, val):
    ref = jax.new_ref(items_emb)
    _get_sc_update()(idx, val, ref)
    return ref[...]
